# baseline (speedup 1.0000x reference)
import functools

import jax
import jax.numpy as jnp
from jax import lax
from jax.experimental import pallas as pl
from jax.experimental.pallas import tpu as pltpu
from jax.experimental.pallas import tpu_sc as plsc

N = 10000
D = 128
E = 320000
NC = 2
NS = 16
NW = NC * NS
EPT = E // NW
K = 80
NCHUNK = EPT // K
NPAD = 10240
RPT = NPAD // NS
R = 400
GRID = N // R

_mesh = plsc.VectorSubcoreMesh(core_axis_name="c", subcore_axis_name="s")


NB = 4
NI = 8


@functools.partial(
    pl.kernel,
    out_type=jax.ShapeDtypeStruct((NC, NPAD, D), jnp.float32),
    mesh=_mesh,
    scratch_types=[
        pltpu.VMEM((K, D), jnp.float32),
        [pltpu.VMEM((K,), jnp.int32) for _ in range(NI)],
        pltpu.VMEM_SHARED((NPAD, D), jnp.float32),
        [pltpu.SemaphoreType.DMA for _ in range(NI)],
        [pltpu.SemaphoreType.DMA for _ in range(NI)],
    ],
)
def _deg_kernel(dst_hbm, ones_hbm, zrows_hbm, out_hbm, ones_v, dd, acc,
                di, ss):
    cid = lax.axis_index("c")
    sid = lax.axis_index("s")
    wid = sid * NC + cid
    base = wid * EPT

    pltpu.sync_copy(ones_hbm, ones_v)
    pltpu.sync_copy(zrows_hbm, acc.at[pl.ds(sid * RPT, RPT)])
    plsc.subcore_barrier()

    def icp(c, s8):
        off = pl.multiple_of(base + c * K, 8)
        return pltpu.make_async_copy(dst_hbm.at[pl.ds(off, K)], dd[s8], di[s8])

    def scp(s8):
        return pltpu.make_async_copy(ones_v, acc.at[dd[s8]], ss[s8])

    def sstart(s8):
        pltpu.async_copy(ones_v, acc.at[dd[s8]], ss[s8], add=True)

    for c in range(4):
        icp(c, c % NI).start()
    for c in range(4):
        icp(c, c % NI).wait()
        sstart(c % NI)
        icp(c + 4, (c + 4) % NI).start()

    def body(c, s8):
        icp(c, s8).wait()
        sstart(s8)
        scp((s8 + 4) % NI).wait()
        icp(c + 4, (s8 + 4) % NI).start()

    def group(j, carry):
        for b2 in range(NI):
            cc = 4 + b2
            body(j * NI + cc, cc % NI)
        return carry

    NGRP = (NCHUNK - 8) // NI
    lax.fori_loop(0, NGRP, group, 0)
    for c in range(4 + NI * NGRP, NCHUNK):
        icp(c, c % NI).wait()
        sstart(c % NI)
        scp((c - 4) % NI).wait()
        if c + 4 < NCHUNK:
            icp(c + 4, (c + 4) % NI).start()
    for c in range(NCHUNK - 4, NCHUNK):
        scp(c % NI).wait()

    plsc.subcore_barrier()
    pltpu.sync_copy(acc.at[pl.ds(sid * RPT, RPT)],
                    out_hbm.at[cid, pl.ds(sid * RPT, RPT)])


@functools.partial(
    pl.kernel,
    out_type=jax.ShapeDtypeStruct((NC, NPAD, D), jnp.float32),
    mesh=_mesh,
    scratch_types=[
        [pltpu.VMEM((K,), jnp.int32) for _ in range(NI)],
        [pltpu.VMEM((K,), jnp.int32) for _ in range(NI)],
        [pltpu.VMEM((K, D), jnp.float32) for _ in range(NB)],
        pltpu.VMEM_SHARED((NPAD, D), jnp.float32),
        [pltpu.SemaphoreType.DMA for _ in range(NI)],
        [pltpu.SemaphoreType.DMA for _ in range(NI)],
        [pltpu.SemaphoreType.DMA for _ in range(NB)],
        [pltpu.SemaphoreType.DMA for _ in range(NB)],
    ],
)
def _agg_kernel(hp_hbm, src_hbm, dst_hbm, zrows_hbm, out_hbm,
                sb, db, rb, acc, si, di, gsem, ssem):
    cid = lax.axis_index("c")
    sid = lax.axis_index("s")
    wid = sid * NC + cid
    base = wid * EPT

    pltpu.sync_copy(zrows_hbm, acc.at[pl.ds(sid * RPT, RPT)])
    plsc.subcore_barrier()

    def icp(c, s8):
        off = pl.multiple_of(base + c * K, 8)
        return (pltpu.make_async_copy(src_hbm.at[pl.ds(off, K)], sb[s8],
                                      si[s8]),
                pltpu.make_async_copy(dst_hbm.at[pl.ds(off, K)], db[s8],
                                      di[s8]))

    def istart(c, s8):
        s, d2 = icp(c, s8)
        s.start()
        d2.start()

    def iwait(c, s8):
        s, d2 = icp(c, s8)
        s.wait()
        d2.wait()

    def gcp(s4, s8):
        return pltpu.make_async_copy(hp_hbm.at[sb[s8]], rb[s4], gsem[s4])

    def scp(s4, s8):
        return pltpu.make_async_copy(rb[s4], acc.at[db[s8]], ssem[s4])

    def sstart(s4, s8):
        pltpu.async_copy(rb[s4], acc.at[db[s8]], ssem[s4], add=True)

    for c in range(4):
        istart(c, c % NI)
    iwait(0, 0)
    gcp(0, 0).start()
    iwait(1, 1)
    gcp(1, 1).start()
    for c in range(2):
        gcp(c % NB, c % NI).wait()
        sstart(c % NB, c % NI)
        iwait(c + 2, (c + 2) % NI)
        gcp((c + 2) % NB, (c + 2) % NI).start()
        istart(c + 4, (c + 4) % NI)

    def body(c, s4, s8):
        gcp(s4, s8).wait()
        sstart(s4, s8)
        iwait(c + 2, (s8 + 2) % NI)
        scp((s4 + 2) % NB, (s8 + 6) % NI).wait()
        gcp((s4 + 2) % NB, (s8 + 2) % NI).start()
        istart(c + 4, (s8 + 4) % NI)

    def group(j, carry):
        for b2 in range(NI):
            cc = 2 + b2
            body(j * NI + cc, cc % NB, cc % NI)
        return carry

    NGRP2 = (120 - 2) // NI
    lax.fori_loop(0, NGRP2, group, 0)
    for c in range(2 + NI * NGRP2, NCHUNK):
        gcp(c % NB, c % NI).wait()
        sstart(c % NB, c % NI)
        if c + 2 < NCHUNK:
            iwait(c + 2, (c + 2) % NI)
        scp((c - 2) % NB, (c - 2) % NI).wait()
        if c + 2 < NCHUNK:
            gcp((c + 2) % NB, (c + 2) % NI).start()
        if c + 4 < NCHUNK:
            istart(c + 4, (c + 4) % NI)
    scp((NCHUNK - 2) % NB, (NCHUNK - 2) % NI).wait()
    scp((NCHUNK - 1) % NB, (NCHUNK - 1) % NI).wait()

    plsc.subcore_barrier()
    pltpu.sync_copy(acc.at[pl.ds(sid * RPT, RPT)],
                    out_hbm.at[cid, pl.ds(sid * RPT, RPT)])


def _dinv_from(deg_blk):
    deg = deg_blk[0, :, 0] + deg_blk[1, :, 0] + 1.0
    return lax.rsqrt(deg)[:, None]


def _tc1_body(x_ref, w1_ref, deg_ref, o_ref):
    dinv = _dinv_from(deg_ref[...])
    h = jnp.dot(x_ref[...], w1_ref[...], preferred_element_type=jnp.float32)
    o_ref[...] = h * dinv


def _tc2_body(agg_ref, hp_ref, deg_ref, b1_ref, w2_ref, o_ref):
    dinv = _dinv_from(deg_ref[...])
    s = agg_ref[0] + agg_ref[1] + hp_ref[...]
    z = jnp.tanh(dinv * s + b1_ref[...])
    o_ref[...] = dinv * jnp.dot(z, w2_ref[...], preferred_element_type=jnp.float32)


def _tc3_body(agg_ref, hp_ref, deg_ref, b2_ref, wf1_ref, bf1_ref, wf2_ref,
              bf2_ref, o_ref):
    dinv = _dinv_from(deg_ref[...])
    s = agg_ref[0] + agg_ref[1] + hp_ref[...]
    z = jnp.tanh(dinv * s + b2_ref[...])
    f = jnp.tanh(jnp.dot(z, wf1_ref[...], preferred_element_type=jnp.float32)
                 + bf1_ref[...])
    o_ref[...] = (jnp.dot(f, wf2_ref[...], preferred_element_type=jnp.float32)
                  + bf2_ref[...])


def _row_spec(width):
    return pl.BlockSpec((R, width), lambda i: (i, 0))


def _full(shape):
    return pl.BlockSpec(shape, lambda i, _s=shape: tuple(0 for _ in _s))


_agg_spec = pl.BlockSpec((NC, R, D), lambda i: (0, i, 0))

_tc1 = pl.pallas_call(
    _tc1_body,
    grid=(GRID,),
    in_specs=[_row_spec(D), _full((D, D)), _agg_spec],
    out_specs=_row_spec(D),
    out_shape=jax.ShapeDtypeStruct((N, D), jnp.float32),
)

_tc2 = pl.pallas_call(
    _tc2_body,
    grid=(GRID,),
    in_specs=[_agg_spec, _row_spec(D), _agg_spec, _full((1, D)),
              _full((D, D))],
    out_specs=_row_spec(D),
    out_shape=jax.ShapeDtypeStruct((N, D), jnp.float32),
)

_tc3 = pl.pallas_call(
    _tc3_body,
    grid=(GRID,),
    in_specs=[_agg_spec, _row_spec(D), _agg_spec, _full((1, D)),
              _full((D, 64)), _full((1, 64)), _full((64, 1)), _full((1, 1))],
    out_specs=_row_spec(1),
    out_shape=jax.ShapeDtypeStruct((N, 1), jnp.float32),
)


def kernel(x, edge_index, W1, b1, W2, b2, Wf1, bf1, Wf2, bf2):
    ei = edge_index.astype(jnp.int32)
    src = ei[0]
    dst = ei[1]
    z128 = jnp.zeros((RPT, D), jnp.float32)

    degp = _deg_kernel(dst, jnp.ones((K, D), jnp.float32), z128)

    h1p = _tc1(x, W1, degp)
    agg1 = _agg_kernel(h1p, src, dst, z128)
    h2p = _tc2(agg1, h1p, degp, b1.reshape(1, D), W2)
    agg2 = _agg_kernel(h2p, src, dst, z128)
    out = _tc3(agg2, h2p, degp, b2.reshape(1, D), Wf1, bf1.reshape(1, 64),
               Wf2, bf2.reshape(1, 1))
    return out


# speedup vs baseline: 1.1169x; 1.1169x over previous
import functools

import jax
import jax.numpy as jnp
from jax import lax
from jax.experimental import pallas as pl
from jax.experimental.pallas import tpu as pltpu
from jax.experimental.pallas import tpu_sc as plsc

N = 10000
D = 128
E = 320000
NC = 2
NS = 16
NW = NC * NS
EPT = E // NW
K = 80
NCHUNK = EPT // K
NPAD = 10240
RPT = NPAD // NS
R = 400
GRID = N // R

NB = 4
NI = 8

_mesh = plsc.VectorSubcoreMesh(core_axis_name="c", subcore_axis_name="s")


@functools.partial(
    pl.kernel,
    out_type=jax.ShapeDtypeStruct((NC, NPAD, D), jnp.float32),
    mesh=_mesh,
    scratch_types=[
        pltpu.VMEM((K, D), jnp.float32),
        [pltpu.VMEM((K,), jnp.int32) for _ in range(NI)],
        pltpu.VMEM_SHARED((NPAD, D), jnp.float32),
        [pltpu.SemaphoreType.DMA for _ in range(NI)],
        [pltpu.SemaphoreType.DMA for _ in range(NI)],
    ],
)
def _deg_kernel(dst_hbm, ones_hbm, zrows_hbm, out_hbm, ones_v, dd, acc,
                di, ss):
    cid = lax.axis_index("c")
    sid = lax.axis_index("s")
    wid = sid * NC + cid
    base = wid * EPT

    pltpu.sync_copy(ones_hbm, ones_v)
    pltpu.sync_copy(zrows_hbm, acc.at[pl.ds(sid * RPT, RPT)])
    plsc.subcore_barrier()

    def icp(c, s8):
        off = pl.multiple_of(base + c * K, 8)
        return pltpu.make_async_copy(dst_hbm.at[pl.ds(off, K)], dd[s8], di[s8])

    def scp(s8):
        return pltpu.make_async_copy(ones_v, acc.at[dd[s8]], ss[s8])

    def sstart(s8):
        pltpu.async_copy(ones_v, acc.at[dd[s8]], ss[s8], add=True)

    for c in range(4):
        icp(c, c % NI).start()
    for c in range(4):
        icp(c, c % NI).wait()
        sstart(c % NI)
        icp(c + 4, (c + 4) % NI).start()

    def body(c, s8):
        icp(c, s8).wait()
        sstart(s8)
        scp((s8 + 4) % NI).wait()
        icp(c + 4, (s8 + 4) % NI).start()

    def group(j, carry):
        for b2 in range(NI):
            cc = 4 + b2
            body(j * NI + cc, cc % NI)
        return carry

    NGRP = (NCHUNK - 8) // NI
    lax.fori_loop(0, NGRP, group, 0)
    for c in range(4 + NI * NGRP, NCHUNK):
        icp(c, c % NI).wait()
        sstart(c % NI)
        scp((c - 4) % NI).wait()
        if c + 4 < NCHUNK:
            icp(c + 4, (c + 4) % NI).start()
    for c in range(NCHUNK - 4, NCHUNK):
        scp(c % NI).wait()

    plsc.subcore_barrier()
    pltpu.sync_copy(acc.at[pl.ds(sid * RPT, RPT)],
                    out_hbm.at[cid, pl.ds(sid * RPT, RPT)])


@functools.partial(
    pl.kernel,
    out_type=jax.ShapeDtypeStruct((NC, NPAD, D), jnp.float32),
    mesh=_mesh,
    scratch_types=[
        [pltpu.VMEM((K,), jnp.int32) for _ in range(NI)],
        [pltpu.VMEM((K,), jnp.int32) for _ in range(NI)],
        [pltpu.VMEM((K, D), jnp.float32) for _ in range(NB)],
        pltpu.VMEM_SHARED((NPAD, D), jnp.float32),
        [pltpu.SemaphoreType.DMA for _ in range(NI)],
        [pltpu.SemaphoreType.DMA for _ in range(NI)],
        [pltpu.SemaphoreType.DMA for _ in range(NB)],
    ],
)
def _agg_kernel(hp_hbm, src_hbm, dst_hbm, zrows_hbm, out_hbm,
                sb, db, rb, acc, si, di, gsem):
    cid = lax.axis_index("c")
    sid = lax.axis_index("s")
    wid = sid * NC + cid
    base = wid * EPT

    pltpu.sync_copy(zrows_hbm, acc.at[pl.ds(sid * RPT, RPT)])
    plsc.subcore_barrier()

    def icp(c, s8):
        off = pl.multiple_of(base + c * K, 8)
        return (pltpu.make_async_copy(src_hbm.at[pl.ds(off, K)], sb[s8],
                                      si[s8]),
                pltpu.make_async_copy(dst_hbm.at[pl.ds(off, K)], db[s8],
                                      di[s8]))

    def istart(c, s8):
        s, d2 = icp(c, s8)
        s.start()
        d2.start()

    def iwait(c, s8):
        s, d2 = icp(c, s8)
        s.wait()
        d2.wait()

    def gcp(s4, s8):
        return pltpu.make_async_copy(hp_hbm.at[sb[s8]], rb[s4], gsem[s4])

    def scatter(s4, s8):
        pltpu.sync_copy(rb[s4], acc.at[db[s8]], add=True)

    for c in range(5):
        istart(c, c % NI)
    for c in range(3):
        iwait(c, c % NI)
        gcp(c % NB, c % NI).start()

    def body(c, s4, s8):
        gcp(s4, s8).wait()
        iwait(c + 3, (s8 + 3) % NI)
        gcp((s4 + 3) % NB, (s8 + 3) % NI).start()
        scatter(s4, s8)
        istart(c + 5, (s8 + 5) % NI)

    def group(j, carry):
        for b2 in range(NI):
            body(j * NI + b2, b2 % NB, b2 % NI)
        return carry

    NGRP2 = 120 // NI
    lax.fori_loop(0, NGRP2, group, 0)
    for c in range(NI * NGRP2, NCHUNK):
        gcp(c % NB, c % NI).wait()
        if c + 3 < NCHUNK:
            iwait(c + 3, (c + 3) % NI)
            gcp((c + 3) % NB, (c + 3) % NI).start()
        scatter(c % NB, c % NI)
        if c + 5 < NCHUNK:
            istart(c + 5, (c + 5) % NI)

    plsc.subcore_barrier()
    pltpu.sync_copy(acc.at[pl.ds(sid * RPT, RPT)],
                    out_hbm.at[cid, pl.ds(sid * RPT, RPT)])


def _dinv_from(deg_blk):
    deg = deg_blk[0, :, 0] + deg_blk[1, :, 0] + 1.0
    return lax.rsqrt(deg)[:, None]


def _tc1_body(x_ref, w1_ref, deg_ref, o_ref):
    dinv = _dinv_from(deg_ref[...])
    h = jnp.dot(x_ref[...], w1_ref[...], preferred_element_type=jnp.float32)
    o_ref[...] = h * dinv


def _tc2_body(agg_ref, hp_ref, deg_ref, b1_ref, w2_ref, o_ref):
    dinv = _dinv_from(deg_ref[...])
    s = agg_ref[0] + agg_ref[1] + hp_ref[...]
    z = jnp.tanh(dinv * s + b1_ref[...])
    o_ref[...] = dinv * jnp.dot(z, w2_ref[...], preferred_element_type=jnp.float32)


def _tc3_body(agg_ref, hp_ref, deg_ref, b2_ref, wf1_ref, bf1_ref, wf2_ref,
              bf2_ref, o_ref):
    dinv = _dinv_from(deg_ref[...])
    s = agg_ref[0] + agg_ref[1] + hp_ref[...]
    z = jnp.tanh(dinv * s + b2_ref[...])
    f = jnp.tanh(jnp.dot(z, wf1_ref[...], preferred_element_type=jnp.float32)
                 + bf1_ref[...])
    o_ref[...] = (jnp.dot(f, wf2_ref[...], preferred_element_type=jnp.float32)
                  + bf2_ref[...])


def _row_spec(width):
    return pl.BlockSpec((R, width), lambda i: (i, 0))


def _full(shape):
    return pl.BlockSpec(shape, lambda i, _s=shape: tuple(0 for _ in _s))


_agg_spec = pl.BlockSpec((NC, R, D), lambda i: (0, i, 0))

_tc1 = pl.pallas_call(
    _tc1_body,
    grid=(GRID,),
    in_specs=[_row_spec(D), _full((D, D)), _agg_spec],
    out_specs=_row_spec(D),
    out_shape=jax.ShapeDtypeStruct((N, D), jnp.float32),
)

_tc2 = pl.pallas_call(
    _tc2_body,
    grid=(GRID,),
    in_specs=[_agg_spec, _row_spec(D), _agg_spec, _full((1, D)),
              _full((D, D))],
    out_specs=_row_spec(D),
    out_shape=jax.ShapeDtypeStruct((N, D), jnp.float32),
)

_tc3 = pl.pallas_call(
    _tc3_body,
    grid=(GRID,),
    in_specs=[_agg_spec, _row_spec(D), _agg_spec, _full((1, D)),
              _full((D, 64)), _full((1, 64)), _full((64, 1)), _full((1, 1))],
    out_specs=_row_spec(1),
    out_shape=jax.ShapeDtypeStruct((N, 1), jnp.float32),
)


def kernel(x, edge_index, W1, b1, W2, b2, Wf1, bf1, Wf2, bf2):
    ei = edge_index.astype(jnp.int32)
    src = ei[0]
    dst = ei[1]
    z128 = jnp.zeros((RPT, D), jnp.float32)

    degp = _deg_kernel(dst, jnp.ones((K, D), jnp.float32), z128)

    h1p = _tc1(x, W1, degp)
    agg1 = _agg_kernel(h1p, src, dst, z128)
    h2p = _tc2(agg1, h1p, degp, b1.reshape(1, D), W2)
    agg2 = _agg_kernel(h2p, src, dst, z128)
    out = _tc3(agg2, h2p, degp, b2.reshape(1, D), Wf1, bf1.reshape(1, 64),
               Wf2, bf2.reshape(1, 1))
    return out
